# baseline (speedup 1.0000x reference)
import jax
import jax.numpy as jnp
from jax.experimental import pallas as pl
from jax.experimental.pallas import tpu as pltpu

_ROWS = 100000
_COLS = 512
_HEAD = 8
_BULK = _ROWS - _HEAD
_N_CHUNKS = 8
_CHUNK = (_BULK // _N_CHUNKS) // 8 * 8
_LAST = _BULK - (_N_CHUNKS - 1) * _CHUNK


def _dma_copy_assign(x_ref, o_ref, head_ref, copy_sems, head_sem):
    copies = []
    start = _HEAD
    for i in range(_N_CHUNKS):
        size = _LAST if i == _N_CHUNKS - 1 else _CHUNK
        c = pltpu.make_async_copy(
            x_ref.at[pl.ds(start, size)],
            o_ref.at[pl.ds(start, size)],
            copy_sems.at[i],
        )
        c.start()
        copies.append(c)
        start += size
    head_in = pltpu.make_async_copy(x_ref.at[pl.ds(0, _HEAD)], head_ref, head_sem)
    head_in.start()
    head_in.wait()
    head_ref[1:3, :] = jnp.ones((2, _COLS), dtype=head_ref.dtype)
    head_out = pltpu.make_async_copy(head_ref, o_ref.at[pl.ds(0, _HEAD)], head_sem)
    head_out.start()
    head_out.wait()
    for c in copies:
        c.wait()


def kernel(x):
    return pl.pallas_call(
        _dma_copy_assign,
        in_specs=[pl.BlockSpec(memory_space=pl.ANY)],
        out_specs=pl.BlockSpec(memory_space=pl.ANY),
        out_shape=jax.ShapeDtypeStruct((_ROWS, _COLS), x.dtype),
        scratch_shapes=[
            pltpu.VMEM((_HEAD, _COLS), x.dtype),
            pltpu.SemaphoreType.DMA((_N_CHUNKS,)),
            pltpu.SemaphoreType.DMA,
        ],
    )(x)


# speedup vs baseline: 44.0999x; 44.0999x over previous
import jax
import jax.numpy as jnp
from jax.experimental import pallas as pl

_ROWS = 100000
_COLS = 512
_BLOCK_ROWS = 1000


def _copy_assign_block(x_ref, o_ref):
    o_ref[...] = x_ref[...]

    @pl.when(pl.program_id(0) == 0)
    def _():
        o_ref[1:3, :] = jnp.ones((2, _COLS), dtype=o_ref.dtype)


def kernel(x):
    grid = _ROWS // _BLOCK_ROWS
    return pl.pallas_call(
        _copy_assign_block,
        grid=(grid,),
        in_specs=[pl.BlockSpec((_BLOCK_ROWS, _COLS), lambda i: (i, 0))],
        out_specs=pl.BlockSpec((_BLOCK_ROWS, _COLS), lambda i: (i, 0)),
        out_shape=jax.ShapeDtypeStruct((_ROWS, _COLS), x.dtype),
    )(x)


# speedup vs baseline: 49.5470x; 1.1235x over previous
import jax
import jax.numpy as jnp
from jax.experimental import pallas as pl

_ROWS = 100000
_COLS = 512
_BLOCK_ROWS = 7000


def _copy_assign_block(x_ref, o_ref):
    o_ref[...] = x_ref[...]

    @pl.when(pl.program_id(0) == 0)
    def _():
        o_ref[1:3, :] = jnp.ones((2, _COLS), dtype=o_ref.dtype)


def kernel(x):
    grid = (_ROWS + _BLOCK_ROWS - 1) // _BLOCK_ROWS
    return pl.pallas_call(
        _copy_assign_block,
        grid=(grid,),
        in_specs=[pl.BlockSpec((_BLOCK_ROWS, _COLS), lambda i: (i, 0))],
        out_specs=pl.BlockSpec((_BLOCK_ROWS, _COLS), lambda i: (i, 0)),
        out_shape=jax.ShapeDtypeStruct((_ROWS, _COLS), x.dtype),
    )(x)
